# baseline (speedup 1.0000x reference)
import functools

import jax
import jax.numpy as jnp
from jax import lax
from jax.experimental import pallas as pl
from jax.experimental.pallas import tpu as pltpu
from jax.experimental.pallas import tpu_sc as plsc

_NC = 2
_NS = 16
_NW = _NC * _NS
_LANES = 16


def _sc_positional_add(tok_flat, pos_flat, B, S, D):
    SD = S * D
    RPW = S // _NW
    T = 8
    TW = T * D
    NT = RPW // T
    NV = TW // _LANES

    mesh = plsc.VectorSubcoreMesh(core_axis_name="c", subcore_axis_name="s")

    @functools.partial(
        pl.kernel,
        mesh=mesh,
        out_type=jax.ShapeDtypeStruct((B * SD,), jnp.float32),
        scratch_types=[
            pltpu.VMEM((TW,), jnp.float32),
            pltpu.VMEM((TW,), jnp.float32),
        ],
    )
    def k(tok_hbm, pos_hbm, out_hbm, pos_buf, tok_buf):
        wid = lax.axis_index("s") * _NC + lax.axis_index("c")
        base = wid * RPW * D

        def tile_body(t, carry):
            off = base + t * TW
            pltpu.sync_copy(pos_hbm.at[pl.ds(off, TW)], pos_buf)
            for b in range(B):
                pltpu.sync_copy(tok_hbm.at[pl.ds(b * SD + off, TW)], tok_buf)

                def vec_body(i, c2):
                    sl = pl.ds(i * _LANES, _LANES)
                    tok_buf[sl] = tok_buf[sl] + pos_buf[sl]
                    return c2

                lax.fori_loop(0, NV, vec_body, 0)
                pltpu.sync_copy(tok_buf, out_hbm.at[pl.ds(b * SD + off, TW)])
            return carry

        lax.fori_loop(0, NT, tile_body, 0)

    return k(tok_flat, pos_flat)


def kernel(token_embeddings, pos_weight):
    B, S, D = token_embeddings.shape
    out_flat = _sc_positional_add(
        token_embeddings.reshape(-1), pos_weight.reshape(-1), B, S, D
    )
    return out_flat.reshape(B, S, D)


# speedup vs baseline: 1.7659x; 1.7659x over previous
import functools

import jax
import jax.numpy as jnp
from jax import lax
from jax.experimental import pallas as pl
from jax.experimental.pallas import tpu as pltpu
from jax.experimental.pallas import tpu_sc as plsc

_NC = 2
_NS = 16
_NW = _NC * _NS
_LANES = 16


def _sc_positional_add(tok_flat, pos_flat, B, S, D):
    SD = S * D
    RPW = S // _NW
    T = 8
    TW = T * D
    NT = RPW // T

    mesh = plsc.VectorSubcoreMesh(core_axis_name="c", subcore_axis_name="s")

    @functools.partial(
        pl.kernel,
        mesh=mesh,
        out_type=jax.ShapeDtypeStruct((B * SD,), jnp.float32),
        scratch_types=[
            [pltpu.VMEM((TW,), jnp.float32) for _ in range(2)],
            [pltpu.VMEM((TW,), jnp.float32) for _ in range(B)],
            [pltpu.SemaphoreType.DMA for _ in range(2)],
            [pltpu.SemaphoreType.DMA for _ in range(B)],
            [pltpu.SemaphoreType.DMA for _ in range(B)],
        ],
    )
    def k(tok_hbm, pos_hbm, out_hbm, pos_bufs, tok_bufs, pos_sems,
          tok_sems, out_sems):
        wid = lax.axis_index("s") * _NC + lax.axis_index("c")
        base = wid * RPW * D

        def pos_src(t):
            return pos_hbm.at[pl.ds(base + t * TW, TW)]

        def tok_src(t, b):
            return tok_hbm.at[pl.ds(b * SD + base + t * TW, TW)]

        def out_dst(t, b):
            return out_hbm.at[pl.ds(b * SD + base + t * TW, TW)]

        pltpu.async_copy(pos_src(0), pos_bufs[0], pos_sems[0])
        for j in range(B):
            pltpu.async_copy(tok_src(0, j), tok_bufs[j], tok_sems[j])

        def pair_body(k3, carry):
            for jj in range(2):
                t = k3 * 2 + jj
                @pl.when(t < NT - 1)
                def _():
                    pltpu.async_copy(pos_src(t + 1), pos_bufs[1 - jj],
                                     pos_sems[1 - jj])

                pltpu.make_async_copy(pos_src(t), pos_bufs[jj],
                                      pos_sems[jj]).wait()

                for j in range(B):
                    pltpu.make_async_copy(tok_src(t, j), tok_bufs[j],
                                          tok_sems[j]).wait()

                    @plsc.parallel_loop(0, TW, step=_LANES, unroll=8)
                    def _(i):
                        sl = pl.ds(i, _LANES)
                        tok_bufs[j][sl] = tok_bufs[j][sl] + pos_bufs[jj][sl]

                    pltpu.async_copy(tok_bufs[j], out_dst(t, j), out_sems[j])

                @pl.when(t < NT - 1)
                def _():
                    for j in range(B):
                        pltpu.make_async_copy(tok_bufs[j], out_dst(t, j),
                                              out_sems[j]).wait()
                        pltpu.async_copy(tok_src(t + 1, j), tok_bufs[j],
                                         tok_sems[j])
            return carry

        lax.fori_loop(0, NT // 2, pair_body, 0)

        for j in range(B):
            pltpu.make_async_copy(tok_bufs[j], out_dst(NT - 1, j),
                                  out_sems[j]).wait()

    return k(tok_flat, pos_flat)


def kernel(token_embeddings, pos_weight):
    B, S, D = token_embeddings.shape
    out_flat = _sc_positional_add(
        token_embeddings.reshape(-1), pos_weight.reshape(-1), B, S, D
    )
    return out_flat.reshape(B, S, D)


# speedup vs baseline: 1.7687x; 1.0016x over previous
import functools

import jax
import jax.numpy as jnp
from jax import lax
from jax.experimental import pallas as pl
from jax.experimental.pallas import tpu as pltpu
from jax.experimental.pallas import tpu_sc as plsc

_NC = 2
_NS = 16
_NW = _NC * _NS
_LANES = 16


def _sc_positional_add(tok_flat, pos_flat, B, S, D):
    SD = S * D
    RPW = S // _NW
    T = 8
    TW = T * D
    NT = RPW // T

    mesh = plsc.VectorSubcoreMesh(core_axis_name="c", subcore_axis_name="s")

    @functools.partial(
        pl.kernel,
        mesh=mesh,
        out_type=jax.ShapeDtypeStruct((B * SD,), jnp.float32),
        scratch_types=[
            [pltpu.VMEM((TW,), jnp.float32) for _ in range(2)],
            [pltpu.VMEM((TW,), jnp.float32) for _ in range(B)],
            [pltpu.SemaphoreType.DMA for _ in range(2)],
            [pltpu.SemaphoreType.DMA for _ in range(B)],
            [pltpu.SemaphoreType.DMA for _ in range(B)],
        ],
    )
    def k(tok_hbm, pos_hbm, out_hbm, pos_bufs, tok_bufs, pos_sems,
          tok_sems, out_sems):
        wid = lax.axis_index("s") * _NC + lax.axis_index("c")
        base = wid * RPW * D

        def pos_src(t):
            return pos_hbm.at[pl.ds(base + t * TW, TW)]

        def tok_src(t, b):
            return tok_hbm.at[pl.ds(b * SD + base + t * TW, TW)]

        def out_dst(t, b):
            return out_hbm.at[pl.ds(b * SD + base + t * TW, TW)]

        pltpu.async_copy(pos_src(0), pos_bufs[0], pos_sems[0])
        for j in range(B):
            pltpu.async_copy(tok_src(0, j), tok_bufs[j], tok_sems[j])

        def pair_body(k3, carry):
            for jj in range(2):
                t = k3 * 2 + jj
                @pl.when(t < NT - 1)
                def _():
                    pltpu.async_copy(pos_src(t + 1), pos_bufs[1 - jj],
                                     pos_sems[1 - jj])

                pltpu.make_async_copy(pos_src(t), pos_bufs[jj],
                                      pos_sems[jj]).wait()

                for j in range(B):
                    pltpu.make_async_copy(tok_src(t, j), tok_bufs[j],
                                          tok_sems[j]).wait()

                    @plsc.parallel_loop(0, TW, step=_LANES, unroll=16)
                    def _(i):
                        sl = pl.ds(i, _LANES)
                        tok_bufs[j][sl] = tok_bufs[j][sl] + pos_bufs[jj][sl]

                    pltpu.async_copy(tok_bufs[j], out_dst(t, j), out_sems[j])

                @pl.when(t < NT - 1)
                def _():
                    for j in range(B):
                        pltpu.make_async_copy(tok_bufs[j], out_dst(t, j),
                                              out_sems[j]).wait()
                        pltpu.async_copy(tok_src(t + 1, j), tok_bufs[j],
                                         tok_sems[j])
            return carry

        lax.fori_loop(0, NT // 2, pair_body, 0)

        for j in range(B):
            pltpu.make_async_copy(tok_bufs[j], out_dst(NT - 1, j),
                                  out_sems[j]).wait()

    return k(tok_flat, pos_flat)


def kernel(token_embeddings, pos_weight):
    B, S, D = token_embeddings.shape
    out_flat = _sc_positional_add(
        token_embeddings.reshape(-1), pos_weight.reshape(-1), B, S, D
    )
    return out_flat.reshape(B, S, D)


# speedup vs baseline: 1.9213x; 1.0863x over previous
import functools

import jax
import jax.numpy as jnp
from jax import lax
from jax.experimental import pallas as pl
from jax.experimental.pallas import tpu as pltpu
from jax.experimental.pallas import tpu_sc as plsc

_NC = 2
_NS = 16
_NW = _NC * _NS
_LANES = 16


def _sc_positional_add(tok_flat, pos_flat, B, S, D):
    SD = S * D
    RPW = S // _NW
    T = 4
    TW = T * D
    NT = RPW // T
    NPAIR = NT // 2

    mesh = plsc.VectorSubcoreMesh(core_axis_name="c", subcore_axis_name="s")

    @functools.partial(
        pl.kernel,
        mesh=mesh,
        out_type=jax.ShapeDtypeStruct((B * SD,), jnp.float32),
        scratch_types=[
            [pltpu.VMEM((TW,), jnp.float32) for _ in range(2)],
            [pltpu.VMEM((TW,), jnp.float32) for _ in range(2 * B)],
            [pltpu.VMEM((TW,), jnp.float32) for _ in range(B)],
            [pltpu.SemaphoreType.DMA for _ in range(2)],
            [pltpu.SemaphoreType.DMA for _ in range(2 * B)],
            [pltpu.SemaphoreType.DMA for _ in range(B)],
        ],
    )
    def k(tok_hbm, pos_hbm, out_hbm, pos_bufs, tok_bufs, res_bufs,
          pos_sems, tok_sems, out_sems):
        wid = lax.axis_index("s") * _NC + lax.axis_index("c")
        base = wid * RPW * D

        def pos_src(t):
            return pos_hbm.at[pl.ds(base + t * TW, TW)]

        def tok_src(t, b):
            return tok_hbm.at[pl.ds(b * SD + base + t * TW, TW)]

        def out_dst(t, b):
            return out_hbm.at[pl.ds(b * SD + base + t * TW, TW)]

        for jj in range(2):
            pltpu.async_copy(pos_src(jj), pos_bufs[jj], pos_sems[jj])
            for b in range(B):
                r = jj * B + b
                pltpu.async_copy(tok_src(jj, b), tok_bufs[r], tok_sems[r])

        def pair_body(k3, carry):
            for jj in range(2):
                t = k3 * 2 + jj
                pltpu.make_async_copy(pos_src(t), pos_bufs[jj],
                                      pos_sems[jj]).wait()
                for b in range(B):
                    r = jj * B + b
                    pltpu.make_async_copy(tok_src(t, b), tok_bufs[r],
                                          tok_sems[r]).wait()

                    @pl.when((k3 > 0) | (jj > 0))
                    def _():
                        pltpu.make_async_copy(res_bufs[b], out_dst(0, 0),
                                              out_sems[b]).wait()

                    @plsc.parallel_loop(0, TW, step=_LANES, unroll=16)
                    def _(i):
                        sl = pl.ds(i, _LANES)
                        res_bufs[b][sl] = tok_bufs[r][sl] + pos_bufs[jj][sl]

                    pltpu.async_copy(res_bufs[b], out_dst(t, b), out_sems[b])

                    @pl.when(k3 < NPAIR - 1)
                    def _():
                        pltpu.async_copy(tok_src(t + 2, b), tok_bufs[r],
                                         tok_sems[r])

                @pl.when(k3 < NPAIR - 1)
                def _():
                    pltpu.async_copy(pos_src(t + 2), pos_bufs[jj],
                                     pos_sems[jj])
            return carry

        lax.fori_loop(0, NPAIR, pair_body, 0)

        for b in range(B):
            pltpu.make_async_copy(res_bufs[b], out_dst(NT - 1, b),
                                  out_sems[b]).wait()

    return k(tok_flat, pos_flat)


def kernel(token_embeddings, pos_weight):
    B, S, D = token_embeddings.shape
    out_flat = _sc_positional_add(
        token_embeddings.reshape(-1), pos_weight.reshape(-1), B, S, D
    )
    return out_flat.reshape(B, S, D)


# speedup vs baseline: 2.0685x; 1.0766x over previous
import functools

import jax
import jax.numpy as jnp
from jax import lax
from jax.experimental import pallas as pl
from jax.experimental.pallas import tpu as pltpu
from jax.experimental.pallas import tpu_sc as plsc

_NC = 2
_NS = 16
_NW = _NC * _NS
_LANES = 16

_S_SC = 1024
_BS_TC = 1024


def _sc_positional_add(tok_flat, pos_flat, B, S, D, S_sc):
    SD = S * D
    OD = S_sc * D
    RPW = S_sc // _NW
    T = 4
    TW = T * D
    NT = RPW // T
    NPAIR = NT // 2

    mesh = plsc.VectorSubcoreMesh(core_axis_name="c", subcore_axis_name="s")

    @functools.partial(
        pl.kernel,
        mesh=mesh,
        out_type=jax.ShapeDtypeStruct((B * OD,), jnp.float32),
        scratch_types=[
            [pltpu.VMEM((TW,), jnp.float32) for _ in range(2)],
            [pltpu.VMEM((TW,), jnp.float32) for _ in range(2 * B)],
            [pltpu.VMEM((TW,), jnp.float32) for _ in range(B)],
            [pltpu.SemaphoreType.DMA for _ in range(2)],
            [pltpu.SemaphoreType.DMA for _ in range(2 * B)],
            [pltpu.SemaphoreType.DMA for _ in range(B)],
        ],
    )
    def sc_add(tok_hbm, pos_hbm, out_hbm, pos_bufs, tok_bufs, res_bufs,
               pos_sems, tok_sems, out_sems):
        wid = lax.axis_index("s") * _NC + lax.axis_index("c")
        base = wid * RPW * D

        def pos_src(t):
            return pos_hbm.at[pl.ds(base + t * TW, TW)]

        def tok_src(t, b):
            return tok_hbm.at[pl.ds(b * SD + base + t * TW, TW)]

        def out_dst(t, b):
            return out_hbm.at[pl.ds(b * OD + base + t * TW, TW)]

        for jj in range(2):
            pltpu.async_copy(pos_src(jj), pos_bufs[jj], pos_sems[jj])
            for b in range(B):
                r = jj * B + b
                pltpu.async_copy(tok_src(jj, b), tok_bufs[r], tok_sems[r])

        def pair_body(k3, carry):
            for jj in range(2):
                t = k3 * 2 + jj
                pltpu.make_async_copy(pos_src(t), pos_bufs[jj],
                                      pos_sems[jj]).wait()
                for b in range(B):
                    r = jj * B + b
                    pltpu.make_async_copy(tok_src(t, b), tok_bufs[r],
                                          tok_sems[r]).wait()

                    @pl.when((k3 > 0) | (jj > 0))
                    def _():
                        pltpu.make_async_copy(res_bufs[b], out_dst(0, 0),
                                              out_sems[b]).wait()

                    @plsc.parallel_loop(0, TW, step=_LANES, unroll=16)
                    def _(i):
                        sl = pl.ds(i, _LANES)
                        res_bufs[b][sl] = tok_bufs[r][sl] + pos_bufs[jj][sl]

                    pltpu.async_copy(res_bufs[b], out_dst(t, b), out_sems[b])

                    @pl.when(k3 < NPAIR - 1)
                    def _():
                        pltpu.async_copy(tok_src(t + 2, b), tok_bufs[r],
                                         tok_sems[r])

                @pl.when(k3 < NPAIR - 1)
                def _():
                    pltpu.async_copy(pos_src(t + 2), pos_bufs[jj],
                                     pos_sems[jj])
            return carry

        lax.fori_loop(0, NPAIR, pair_body, 0)

        for b in range(B):
            pltpu.make_async_copy(res_bufs[b], out_dst(NT - 1, b),
                                  out_sems[b]).wait()

    return sc_add(tok_flat, pos_flat)


def _tc_body(tok_ref, pos_ref, out_ref):
    out_ref[...] = tok_ref[...] + pos_ref[...]


def _tc_positional_add(tok, pos, B, S, D, S_off):
    S_tc = S - S_off
    off_blocks = S_off // _BS_TC
    return pl.pallas_call(
        _tc_body,
        grid=(S_tc // _BS_TC, B),
        in_specs=[
            pl.BlockSpec((1, _BS_TC, D), lambda s, b: (b, s + off_blocks, 0)),
            pl.BlockSpec((_BS_TC, D), lambda s, b: (s + off_blocks, 0)),
        ],
        out_specs=pl.BlockSpec((1, _BS_TC, D), lambda s, b: (b, s, 0)),
        out_shape=jax.ShapeDtypeStruct((B, S_tc, D), tok.dtype),
    )(tok, pos)


def kernel(token_embeddings, pos_weight):
    B, S, D = token_embeddings.shape
    sc_out = _sc_positional_add(
        token_embeddings.reshape(-1), pos_weight.reshape(-1), B, S, D, _S_SC
    ).reshape(B, _S_SC, D)
    tc_out = _tc_positional_add(token_embeddings, pos_weight, B, S, D, _S_SC)
    return jnp.concatenate([sc_out, tc_out], axis=1)


# speedup vs baseline: 7.9065x; 3.8223x over previous
import jax
import jax.numpy as jnp
from jax.experimental import pallas as pl


def _add_body(tok_ref, pos_ref, out_ref):
    out_ref[...] = tok_ref[...] + pos_ref[...]


def kernel(token_embeddings, pos_weight):
    B, S, D = token_embeddings.shape
    BS = 1024
    grid = (S // BS, B)
    return pl.pallas_call(
        _add_body,
        grid=grid,
        in_specs=[
            pl.BlockSpec((1, BS, D), lambda s, b: (b, s, 0)),
            pl.BlockSpec((BS, D), lambda s, b: (s, 0)),
        ],
        out_specs=pl.BlockSpec((1, BS, D), lambda s, b: (b, s, 0)),
        out_shape=jax.ShapeDtypeStruct((B, S, D), token_embeddings.dtype),
    )(token_embeddings, pos_weight)
